# Initial kernel scaffold; baseline (speedup 1.0000x reference)
import jax
import jax.numpy as jnp
from jax.experimental import pallas as pl


def kernel(voxels, voxel_num_points, voxel_coords, edge_index, params):
    raise NotImplementedError("write your pallas kernel here")


# speedup vs baseline: 3.3152x; 3.3152x over previous
import functools

import jax
import jax.numpy as jnp
from jax import lax
from jax.experimental import pallas as pl
from jax.experimental.pallas import tpu as pltpu
from jax.experimental.pallas import tpu_sc as plsc

N = 50000
E = 800000
ACC = 50048
BLK = 2000
NB = N // BLK
CS_J = 400
ES_J = 200
IB = 40


def _make_edge_agg(J, R):
    mesh = plsc.VectorSubcoreMesh(core_axis_name="c", subcore_axis_name="s",
                                  num_cores=2, num_subcores=16)
    ZR = 256
    stripe = ACC // 16
    nfull, rem = divmod(stripe, ZR)

    @functools.partial(
        pl.kernel,
        mesh=mesh,
        compiler_params=pltpu.CompilerParams(use_tc_tiling_on_sc=False),
        out_type=jax.ShapeDtypeStruct((2, ACC, 16), jnp.float32),
        scratch_types=[
            pltpu.VMEM((IB, 128), jnp.int32),
            pltpu.VMEM((IB, 128), jnp.int32),
            pltpu.VMEM((128, 16), jnp.float32),
            pltpu.VMEM((ZR, 16), jnp.float32),
            pltpu.VMEM_SHARED((ACC, 16), jnp.float32),
            pltpu.SemaphoreType.DMA,
        ],
    )
    def k(table_hbm, src_hbm, dst_hbm, out_hbm, src_v, dst_v, rows_v, zbuf,
          acc, sem):
        c = lax.axis_index("c")
        s = lax.axis_index("s")

        def zfill(i, carry):
            zbuf[i, pl.ds(0, 16)] = jnp.zeros((16,), jnp.float32)
            return carry

        lax.fori_loop(0, ZR, zfill, 0)
        base = s * stripe
        for q in range(nfull):
            pltpu.sync_copy(zbuf, acc.at[pl.ds(base + q * ZR, ZR)])
        if rem:
            pltpu.sync_copy(zbuf.at[pl.ds(0, rem)],
                            acc.at[pl.ds(base + nfull * ZR, rem)])
        plsc.subcore_barrier()

        def outer(b, carry):
            pltpu.sync_copy(src_hbm.at[c, s, pl.ds(b * IB, IB)], src_v)
            pltpu.sync_copy(dst_hbm.at[c, s, pl.ds(b * IB, IB)], dst_v)

            def body(j, carry2):
                pltpu.async_copy(table_hbm.at[src_v.at[j]], rows_v,
                                 sem).wait()
                pltpu.sync_copy(rows_v, acc.at[dst_v.at[j]], add=True)
                return carry2

            lax.fori_loop(0, IB, body, 0)
            return carry

        lax.fori_loop(0, J // IB, outer, 0)
        plsc.subcore_barrier()
        pltpu.sync_copy(acc.at[pl.ds(base, stripe)],
                        out_hbm.at[c, pl.ds(base, stripe)])

    return k


def _make_k1(cins, cout, scaled=False):
    ni = len(cins)
    nq = cout // 16

    def body(*refs):
        xs, ws = refs[:ni], refs[ni:2 * ni]
        rest = refs[2 * ni:]
        if scaled:
            scale, out = rest[0], rest[1]
        else:
            out = rest[0]
        acc = jnp.dot(xs[0][...], ws[0][0],
                      preferred_element_type=jnp.float32)
        for t in range(1, ni):
            acc = acc + jnp.dot(xs[t][...], ws[t][0],
                                preferred_element_type=jnp.float32)
        if scaled:
            acc = acc / jnp.clip(scale[...], 1.0, None)
        out[0] = acc

    grid = (nq, NB)
    in_specs = (
        [pl.BlockSpec((BLK, ci), lambda q, i: (i, 0)) for ci in cins]
        + [pl.BlockSpec((1, ci, 16), lambda q, i: (q, 0, 0)) for ci in cins]
        + ([pl.BlockSpec((BLK, 1), lambda q, i: (i, 0))] if scaled else [])
    )
    out_spec = pl.BlockSpec((1, BLK, 16), lambda q, i: (q, i, 0))
    out_shape = jax.ShapeDtypeStruct((nq, N, 16), jnp.float32)
    return pl.pallas_call(body, grid=grid, in_specs=in_specs,
                          out_specs=out_spec, out_shape=out_shape)


def _make_k3(cout, variant):
    nq = cout // 16
    n_agg = 2 if cout == 64 else 1
    n_extra = {"plain": 0, "resid": 1, "ur": 2}[variant]

    def body(*refs):
        y_ref = refs[0]
        a_refs = refs[1:1 + n_agg]
        extra = refs[1 + n_agg:1 + n_agg + n_extra]
        out, s1, s2 = refs[-3], refs[-2], refs[-1]
        p = pl.program_id(0)
        i = pl.program_id(1)
        yb = y_ref[...]
        if cout == 16:
            a = a_refs[0][...]
            t = yb[0] + a[0] + a[1]
        else:
            parts = []
            for q in range(nq):
                a = a_refs[q // 2][...]
                parts.append(yb[q] + a[q % 2])
            t = jnp.concatenate(parts, axis=1)

        @pl.when(p == 0)
        def _():
            ps1 = jnp.sum(t, axis=0, keepdims=True)
            ps2 = jnp.sum(t * t, axis=0, keepdims=True)
            first = i == 0
            s1[...] = jnp.where(first, ps1, s1[...] + ps1)
            s2[...] = jnp.where(first, ps2, s2[...] + ps2)

        @pl.when(p == 1)
        def _():
            m = s1[...] / N
            v = s2[...] / N - m * m
            xn = (t - m) * lax.rsqrt(v + 1e-3)
            if variant == "plain":
                r = jnp.maximum(xn, 0.0)
            elif variant == "resid":
                r = jnp.maximum(xn + extra[0][...], 0.0)
            else:
                rows = lax.broadcasted_iota(jnp.int32, (cout, cout // 2), 0)
                cols = lax.broadcasted_iota(jnp.int32, (cout, cout // 2), 1)
                pm = (rows // 2 == cols).astype(jnp.float32)
                red = jnp.concatenate(
                    [jnp.dot(extra[0][...], pm,
                             preferred_element_type=jnp.float32),
                     jnp.dot(extra[1][...], pm,
                             preferred_element_type=jnp.float32)], axis=1)
                r = jnp.maximum(xn, 0.0) + red
            out[...] = r

    in_specs = (
        [pl.BlockSpec((nq, BLK, 16), lambda p, i: (0, i, 0))]
        + [pl.BlockSpec((2, BLK, 16), lambda p, i: (0, i, 0))] * n_agg
        + [pl.BlockSpec((BLK, cout), lambda p, i: (i, 0))] * n_extra
    )
    return pl.pallas_call(
        body,
        grid=(2, NB),
        in_specs=in_specs,
        out_specs=pl.BlockSpec((BLK, cout), lambda p, i: (i, 0)),
        out_shape=jax.ShapeDtypeStruct((N, cout), jnp.float32),
        scratch_shapes=[pltpu.VMEM((1, cout), jnp.float32)] * 2,
    )


def _pack_cs(src, dst):
    s = jnp.pad(src.reshape(16, 50000), ((0, 0), (0, 1200)))
    d = jnp.pad(dst.reshape(16, 50000), ((0, 0), (0, 1200)),
                constant_values=N)
    s = s.reshape(16, CS_J, 128)
    d = d.reshape(16, CS_J, 128)
    return jnp.stack([s, s + N]), jnp.stack([d, d])


def _pack_es(src, dst):
    s = jnp.pad(src.reshape(2, 16, 25000), ((0, 0), (0, 0), (0, 600)))
    d = jnp.pad(dst.reshape(2, 16, 25000), ((0, 0), (0, 0), (0, 600)),
                constant_values=N)
    return s.reshape(2, 16, ES_J, 128), d.reshape(2, 16, ES_J, 128)


def kernel(voxels, voxel_num_points, voxel_coords, edge_index, params):
    p = params
    src = edge_index[0].astype(jnp.int32)
    dst = edge_index[1].astype(jnp.int32)
    packs = {"cs": _pack_cs(src, dst), "es": _pack_es(src, dst)}
    agg_cs = _make_edge_agg(CS_J, 2 * N)
    agg_es = _make_edge_agg(ES_J, N)

    def conv(xs, ws, variant="plain", extras=(), scale=None):
        cout = ws[0].shape[1]
        nq = cout // 16
        cins = tuple(x.shape[1] for x in xs)
        wstk = [jnp.stack([w[:, 16 * q:16 * (q + 1)] for q in range(nq)])
                for w in ws]
        k1 = _make_k1(cins, cout, scaled=scale is not None)
        args = list(xs) + wstk + ([scale] if scale is not None else [])
        table = k1(*args)
        if cout == 16:
            spack, dpack = packs["es"]
            aggs = [agg_es(table.reshape(N, 16), spack, dpack)]
        else:
            spack, dpack = packs["cs"]
            aggs = [agg_cs(table[2 * h:2 * h + 2].reshape(2 * N, 16),
                           spack, dpack) for h in range(nq // 2)]
        k3 = _make_k3(cout, variant)
        return k3(table, *aggs, *extras)

    def block(x, w):
        return conv([x], [w])

    def basic(x, wa, wb):
        h = conv([x], [wa])
        return conv([h], [wb], variant="resid", extras=(x,))

    def ur(x_lat, x_bot, wta, wtb, wm, winv):
        xt = basic(x_lat, wta, wtb)
        c = xt.shape[1]
        xm = conv([x_bot, xt], [wm[:c], wm[c:]], variant="ur",
                  extras=(x_bot, xt))
        return conv([xm], [winv])

    xv = voxels.reshape(N, 20)
    w_rep = jnp.concatenate([p['W_in']] * 5, axis=0)
    cnt = voxel_num_points.astype(jnp.float32).reshape(N, 1)
    x = conv([xv], [w_rep], scale=cnt)

    x1 = block(x, p['W1'])
    x2 = block(block(block(x1, p['W2a']), p['W2b']), p['W2c'])
    x3 = block(block(block(x2, p['W3a']), p['W3b']), p['W3c'])
    x4 = block(block(block(x3, p['W4a']), p['W4b']), p['W4c'])
    up4 = ur(x4, x4, p['Wt4a'], p['Wt4b'], p['Wm4'], p['Winv4'])
    up3 = ur(x3, up4, p['Wt3a'], p['Wt3b'], p['Wm3'], p['Winv3'])
    up2 = ur(x2, up3, p['Wt2a'], p['Wt2b'], p['Wm2'], p['Winv2'])
    up1 = ur(x1, up2, p['Wt1a'], p['Wt1b'], p['Wm1'], p['Wc5'])
    return up1


# speedup vs baseline: 5.0527x; 1.5241x over previous
import functools

import jax
import jax.numpy as jnp
from jax import lax
from jax.experimental import pallas as pl
from jax.experimental.pallas import tpu as pltpu
from jax.experimental.pallas import tpu_sc as plsc

N = 50000
E = 800000
ACC = 50048
BLK = 2000
NB = N // BLK
CS_J = 400
ES_J = 200
IB = 40


NBUF = 4


def _make_edge_agg(J, R):
    mesh = plsc.VectorSubcoreMesh(core_axis_name="c", subcore_axis_name="s",
                                  num_cores=2, num_subcores=16)
    ZR = 256
    stripe = ACC // 16
    nfull, rem = divmod(stripe, ZR)

    @functools.partial(
        pl.kernel,
        mesh=mesh,
        compiler_params=pltpu.CompilerParams(use_tc_tiling_on_sc=False),
        out_type=jax.ShapeDtypeStruct((2, ACC, 16), jnp.float32),
        scratch_types=[
            pltpu.VMEM((IB, 128), jnp.int32),
            pltpu.VMEM((IB, 128), jnp.int32),
            pltpu.VMEM((NBUF, 128, 16), jnp.float32),
            pltpu.VMEM((ZR, 16), jnp.float32),
            pltpu.VMEM_SHARED((ACC, 16), jnp.float32),
        ] + [pltpu.SemaphoreType.DMA] * NBUF,
    )
    def k(table_hbm, src_hbm, dst_hbm, out_hbm, src_v, dst_v, rows_v, zbuf,
          acc, *sems):
        c = lax.axis_index("c")
        s = lax.axis_index("s")

        def zfill(i, carry):
            zbuf[i, pl.ds(0, 16)] = jnp.zeros((16,), jnp.float32)
            return carry

        lax.fori_loop(0, ZR, zfill, 0)
        base = s * stripe
        for q in range(nfull):
            pltpu.sync_copy(zbuf, acc.at[pl.ds(base + q * ZR, ZR)])
        if rem:
            pltpu.sync_copy(zbuf.at[pl.ds(0, rem)],
                            acc.at[pl.ds(base + nfull * ZR, rem)])
        plsc.subcore_barrier()

        def outer(blk, carry):
            pltpu.sync_copy(src_hbm.at[c, s, pl.ds(blk * IB, IB)], src_v)
            pltpu.sync_copy(dst_hbm.at[c, s, pl.ds(blk * IB, IB)], dst_v)

            for b in range(NBUF):
                pltpu.async_copy(table_hbm.at[src_v.at[b]], rows_v.at[b],
                                 sems[b])

            def body(t, carry2):
                g = t * NBUF
                for b in range(NBUF):
                    j = g + b
                    pltpu.make_async_copy(table_hbm.at[src_v.at[0]],
                                          rows_v.at[b], sems[b]).wait()
                    pltpu.sync_copy(rows_v.at[b], acc.at[dst_v.at[j]],
                                    add=True)
                    pltpu.async_copy(table_hbm.at[src_v.at[j + NBUF]],
                                     rows_v.at[b], sems[b])
                return carry2

            lax.fori_loop(0, IB // NBUF - 1, body, 0)
            for b in range(NBUF):
                j = IB - NBUF + b
                pltpu.make_async_copy(table_hbm.at[src_v.at[0]],
                                      rows_v.at[b], sems[b]).wait()
                pltpu.sync_copy(rows_v.at[b], acc.at[dst_v.at[j]], add=True)
            return carry

        lax.fori_loop(0, J // IB, outer, 0)
        plsc.subcore_barrier()
        pltpu.sync_copy(acc.at[pl.ds(base, stripe)],
                        out_hbm.at[c, pl.ds(base, stripe)])

    return k


def _make_k1(cins, cout, scaled=False):
    ni = len(cins)
    nq = cout // 16

    def body(*refs):
        xs, ws = refs[:ni], refs[ni:2 * ni]
        rest = refs[2 * ni:]
        if scaled:
            scale, out = rest[0], rest[1]
        else:
            out = rest[0]
        acc = jnp.dot(xs[0][...], ws[0][0],
                      preferred_element_type=jnp.float32)
        for t in range(1, ni):
            acc = acc + jnp.dot(xs[t][...], ws[t][0],
                                preferred_element_type=jnp.float32)
        if scaled:
            acc = acc / jnp.clip(scale[...], 1.0, None)
        out[0] = acc

    grid = (nq, NB)
    in_specs = (
        [pl.BlockSpec((BLK, ci), lambda q, i: (i, 0)) for ci in cins]
        + [pl.BlockSpec((1, ci, 16), lambda q, i: (q, 0, 0)) for ci in cins]
        + ([pl.BlockSpec((BLK, 1), lambda q, i: (i, 0))] if scaled else [])
    )
    out_spec = pl.BlockSpec((1, BLK, 16), lambda q, i: (q, i, 0))
    out_shape = jax.ShapeDtypeStruct((nq, N, 16), jnp.float32)
    return pl.pallas_call(body, grid=grid, in_specs=in_specs,
                          out_specs=out_spec, out_shape=out_shape)


def _make_k3(cout, variant):
    nq = cout // 16
    n_agg = 2 if cout == 64 else 1
    n_extra = {"plain": 0, "resid": 1, "ur": 2}[variant]

    def body(*refs):
        y_ref = refs[0]
        a_refs = refs[1:1 + n_agg]
        extra = refs[1 + n_agg:1 + n_agg + n_extra]
        out, s1, s2 = refs[-3], refs[-2], refs[-1]
        p = pl.program_id(0)
        i = pl.program_id(1)
        yb = y_ref[...]
        if cout == 16:
            a = a_refs[0][...]
            t = yb[0] + a[0] + a[1]
        else:
            parts = []
            for q in range(nq):
                a = a_refs[q // 2][...]
                parts.append(yb[q] + a[q % 2])
            t = jnp.concatenate(parts, axis=1)

        @pl.when(p == 0)
        def _():
            ps1 = jnp.sum(t, axis=0, keepdims=True)
            ps2 = jnp.sum(t * t, axis=0, keepdims=True)
            first = i == 0
            s1[...] = jnp.where(first, ps1, s1[...] + ps1)
            s2[...] = jnp.where(first, ps2, s2[...] + ps2)

        @pl.when(p == 1)
        def _():
            m = s1[...] / N
            v = s2[...] / N - m * m
            xn = (t - m) * lax.rsqrt(v + 1e-3)
            if variant == "plain":
                r = jnp.maximum(xn, 0.0)
            elif variant == "resid":
                r = jnp.maximum(xn + extra[0][...], 0.0)
            else:
                rows = lax.broadcasted_iota(jnp.int32, (cout, cout // 2), 0)
                cols = lax.broadcasted_iota(jnp.int32, (cout, cout // 2), 1)
                pm = (rows // 2 == cols).astype(jnp.float32)
                red = jnp.concatenate(
                    [jnp.dot(extra[0][...], pm,
                             preferred_element_type=jnp.float32),
                     jnp.dot(extra[1][...], pm,
                             preferred_element_type=jnp.float32)], axis=1)
                r = jnp.maximum(xn, 0.0) + red
            out[...] = r

    in_specs = (
        [pl.BlockSpec((nq, BLK, 16), lambda p, i: (0, i, 0))]
        + [pl.BlockSpec((2, BLK, 16), lambda p, i: (0, i, 0))] * n_agg
        + [pl.BlockSpec((BLK, cout), lambda p, i: (i, 0))] * n_extra
    )
    return pl.pallas_call(
        body,
        grid=(2, NB),
        in_specs=in_specs,
        out_specs=pl.BlockSpec((BLK, cout), lambda p, i: (i, 0)),
        out_shape=jax.ShapeDtypeStruct((N, cout), jnp.float32),
        scratch_shapes=[pltpu.VMEM((1, cout), jnp.float32)] * 2,
    )


def _pack_cs(src, dst):
    s = jnp.pad(src.reshape(16, 50000), ((0, 0), (0, 1200)))
    d = jnp.pad(dst.reshape(16, 50000), ((0, 0), (0, 1200)),
                constant_values=N)
    s = s.reshape(16, CS_J, 128)
    d = d.reshape(16, CS_J, 128)
    return jnp.stack([s, s + N]), jnp.stack([d, d])


def _pack_es(src, dst):
    s = jnp.pad(src.reshape(2, 16, 25000), ((0, 0), (0, 0), (0, 600)))
    d = jnp.pad(dst.reshape(2, 16, 25000), ((0, 0), (0, 0), (0, 600)),
                constant_values=N)
    return s.reshape(2, 16, ES_J, 128), d.reshape(2, 16, ES_J, 128)


def kernel(voxels, voxel_num_points, voxel_coords, edge_index, params):
    p = params
    src = edge_index[0].astype(jnp.int32)
    dst = edge_index[1].astype(jnp.int32)
    packs = {"cs": _pack_cs(src, dst), "es": _pack_es(src, dst)}
    agg_cs = _make_edge_agg(CS_J, 2 * N)
    agg_es = _make_edge_agg(ES_J, N)

    def conv(xs, ws, variant="plain", extras=(), scale=None):
        cout = ws[0].shape[1]
        nq = cout // 16
        cins = tuple(x.shape[1] for x in xs)
        wstk = [jnp.stack([w[:, 16 * q:16 * (q + 1)] for q in range(nq)])
                for w in ws]
        k1 = _make_k1(cins, cout, scaled=scale is not None)
        args = list(xs) + wstk + ([scale] if scale is not None else [])
        table = k1(*args)
        if cout == 16:
            spack, dpack = packs["es"]
            aggs = [agg_es(table.reshape(N, 16), spack, dpack)]
        else:
            spack, dpack = packs["cs"]
            aggs = [agg_cs(table[2 * h:2 * h + 2].reshape(2 * N, 16),
                           spack, dpack) for h in range(nq // 2)]
        k3 = _make_k3(cout, variant)
        return k3(table, *aggs, *extras)

    def block(x, w):
        return conv([x], [w])

    def basic(x, wa, wb):
        h = conv([x], [wa])
        return conv([h], [wb], variant="resid", extras=(x,))

    def ur(x_lat, x_bot, wta, wtb, wm, winv):
        xt = basic(x_lat, wta, wtb)
        c = xt.shape[1]
        xm = conv([x_bot, xt], [wm[:c], wm[c:]], variant="ur",
                  extras=(x_bot, xt))
        return conv([xm], [winv])

    xv = voxels.reshape(N, 20)
    w_rep = jnp.concatenate([p['W_in']] * 5, axis=0)
    cnt = voxel_num_points.astype(jnp.float32).reshape(N, 1)
    x = conv([xv], [w_rep], scale=cnt)

    x1 = block(x, p['W1'])
    x2 = block(block(block(x1, p['W2a']), p['W2b']), p['W2c'])
    x3 = block(block(block(x2, p['W3a']), p['W3b']), p['W3c'])
    x4 = block(block(block(x3, p['W4a']), p['W4b']), p['W4c'])
    up4 = ur(x4, x4, p['Wt4a'], p['Wt4b'], p['Wm4'], p['Winv4'])
    up3 = ur(x3, up4, p['Wt3a'], p['Wt3b'], p['Wm3'], p['Winv3'])
    up2 = ur(x2, up3, p['Wt2a'], p['Wt2b'], p['Wm2'], p['Winv2'])
    up1 = ur(x1, up2, p['Wt1a'], p['Wt1b'], p['Wm1'], p['Wc5'])
    return up1


# speedup vs baseline: 6.4474x; 1.2760x over previous
import functools

import jax
import jax.numpy as jnp
from jax import lax
from jax.experimental import pallas as pl
from jax.experimental.pallas import tpu as pltpu
from jax.experimental.pallas import tpu_sc as plsc

N = 50000
E = 800000
ACC = 50048
BLK = 2000
NB = N // BLK
CS_J = 400
ES_J = 200
IB = 40


NBUF = 4


def _make_edge_agg(J, R, C, ib):
    mesh = plsc.VectorSubcoreMesh(core_axis_name="c", subcore_axis_name="s",
                                  num_cores=2, num_subcores=16)
    ZR = 128
    IB = ib
    stripe = ACC // 16
    nfull, rem = divmod(stripe, ZR)

    @functools.partial(
        pl.kernel,
        mesh=mesh,
        compiler_params=pltpu.CompilerParams(use_tc_tiling_on_sc=False),
        out_type=jax.ShapeDtypeStruct((2, ACC, C), jnp.float32),
        scratch_types=[
            pltpu.VMEM((IB, 128), jnp.int32),
            pltpu.VMEM((IB, 128), jnp.int32),
            pltpu.VMEM((NBUF, 128, C), jnp.float32),
            pltpu.VMEM((ZR, C), jnp.float32),
            pltpu.VMEM_SHARED((ACC, C), jnp.float32),
        ] + [pltpu.SemaphoreType.DMA] * NBUF,
    )
    def k(table_hbm, src_hbm, dst_hbm, out_hbm, src_v, dst_v, rows_v, zbuf,
          acc, *sems):
        c = lax.axis_index("c")
        s = lax.axis_index("s")

        def zfill(i, carry):
            for g in range(C // 16):
                zbuf[i, pl.ds(16 * g, 16)] = jnp.zeros((16,), jnp.float32)
            return carry

        lax.fori_loop(0, ZR, zfill, 0)
        base = s * stripe
        for q in range(nfull):
            pltpu.sync_copy(zbuf, acc.at[pl.ds(base + q * ZR, ZR)])
        if rem:
            pltpu.sync_copy(zbuf.at[pl.ds(0, rem)],
                            acc.at[pl.ds(base + nfull * ZR, rem)])
        plsc.subcore_barrier()

        def outer(blk, carry):
            pltpu.sync_copy(src_hbm.at[c, s, pl.ds(blk * IB, IB)], src_v)
            pltpu.sync_copy(dst_hbm.at[c, s, pl.ds(blk * IB, IB)], dst_v)

            for b in range(NBUF):
                pltpu.async_copy(table_hbm.at[src_v.at[b]], rows_v.at[b],
                                 sems[b])

            def body(t, carry2):
                g = t * NBUF
                for b in range(NBUF):
                    j = g + b
                    pltpu.make_async_copy(table_hbm.at[src_v.at[0]],
                                          rows_v.at[b], sems[b]).wait()
                    pltpu.sync_copy(rows_v.at[b], acc.at[dst_v.at[j]],
                                    add=True)
                    pltpu.async_copy(table_hbm.at[src_v.at[j + NBUF]],
                                     rows_v.at[b], sems[b])
                return carry2

            lax.fori_loop(0, IB // NBUF - 1, body, 0)
            for b in range(NBUF):
                j = IB - NBUF + b
                pltpu.make_async_copy(table_hbm.at[src_v.at[0]],
                                      rows_v.at[b], sems[b]).wait()
                pltpu.sync_copy(rows_v.at[b], acc.at[dst_v.at[j]], add=True)
            return carry

        lax.fori_loop(0, J // IB, outer, 0)
        plsc.subcore_barrier()
        pltpu.sync_copy(acc.at[pl.ds(base, stripe)],
                        out_hbm.at[c, pl.ds(base, stripe)])

    return k


def _make_k1(cins, cout, scaled=False):
    ni = len(cins)
    G = 16 if cout == 16 else 32
    nq = cout // G

    def body(*refs):
        xs, ws = refs[:ni], refs[ni:2 * ni]
        rest = refs[2 * ni:]
        if scaled:
            scale, out = rest[0], rest[1]
        else:
            out = rest[0]
        acc = jnp.dot(xs[0][...], ws[0][0],
                      preferred_element_type=jnp.float32)
        for t in range(1, ni):
            acc = acc + jnp.dot(xs[t][...], ws[t][0],
                                preferred_element_type=jnp.float32)
        if scaled:
            acc = acc / jnp.clip(scale[...], 1.0, None)
        out[0] = acc

    grid = (nq, NB)
    in_specs = (
        [pl.BlockSpec((BLK, ci), lambda q, i: (i, 0)) for ci in cins]
        + [pl.BlockSpec((1, ci, G), lambda q, i: (q, 0, 0)) for ci in cins]
        + ([pl.BlockSpec((BLK, 1), lambda q, i: (i, 0))] if scaled else [])
    )
    out_spec = pl.BlockSpec((1, BLK, G), lambda q, i: (q, i, 0))
    out_shape = jax.ShapeDtypeStruct((nq, N, G), jnp.float32)
    return pl.pallas_call(body, grid=grid, in_specs=in_specs,
                          out_specs=out_spec, out_shape=out_shape)


def _make_k3(cout, variant):
    G = 16 if cout == 16 else 32
    nq = cout // G
    n_extra = {"plain": 0, "resid": 1, "ur": 2}[variant]

    def body(*refs):
        y_ref = refs[0]
        a_ref = refs[1]
        extra = refs[2:2 + n_extra]
        out, s1, s2 = refs[-3], refs[-2], refs[-1]
        p = pl.program_id(0)
        i = pl.program_id(1)
        yb = y_ref[...]
        a = a_ref[...]
        if nq == 1:
            t = yb[0] + a[0] + a[1]
        else:
            t = jnp.concatenate([yb[q] + a[q] for q in range(nq)], axis=1)

        @pl.when(p == 0)
        def _():
            ps1 = jnp.sum(t, axis=0, keepdims=True)
            ps2 = jnp.sum(t * t, axis=0, keepdims=True)
            first = i == 0
            s1[...] = jnp.where(first, ps1, s1[...] + ps1)
            s2[...] = jnp.where(first, ps2, s2[...] + ps2)

        @pl.when(p == 1)
        def _():
            m = s1[...] / N
            v = s2[...] / N - m * m
            xn = (t - m) * lax.rsqrt(v + 1e-3)
            if variant == "plain":
                r = jnp.maximum(xn, 0.0)
            elif variant == "resid":
                r = jnp.maximum(xn + extra[0][...], 0.0)
            else:
                rows = lax.broadcasted_iota(jnp.int32, (cout, cout // 2), 0)
                cols = lax.broadcasted_iota(jnp.int32, (cout, cout // 2), 1)
                pm = (rows // 2 == cols).astype(jnp.float32)
                red = jnp.concatenate(
                    [jnp.dot(extra[0][...], pm,
                             preferred_element_type=jnp.float32),
                     jnp.dot(extra[1][...], pm,
                             preferred_element_type=jnp.float32)], axis=1)
                r = jnp.maximum(xn, 0.0) + red
            out[...] = r

    in_specs = (
        [pl.BlockSpec((nq, BLK, G), lambda p, i: (0, i, 0))]
        + [pl.BlockSpec((2, BLK, G), lambda p, i: (0, i, 0))]
        + [pl.BlockSpec((BLK, cout), lambda p, i: (i, 0))] * n_extra
    )
    return pl.pallas_call(
        body,
        grid=(2, NB),
        in_specs=in_specs,
        out_specs=pl.BlockSpec((BLK, cout), lambda p, i: (i, 0)),
        out_shape=jax.ShapeDtypeStruct((N, cout), jnp.float32),
        scratch_shapes=[pltpu.VMEM((1, cout), jnp.float32)] * 2,
    )


def _pack_cs(src, dst):
    s = jnp.pad(src.reshape(16, 50000), ((0, 0), (0, 1200)))
    d = jnp.pad(dst.reshape(16, 50000), ((0, 0), (0, 1200)),
                constant_values=N)
    s = s.reshape(16, CS_J, 128)
    d = d.reshape(16, CS_J, 128)
    return jnp.stack([s, s + N]), jnp.stack([d, d])


def _pack_es(src, dst):
    s = jnp.pad(src.reshape(2, 16, 25000), ((0, 0), (0, 0), (0, 600)))
    d = jnp.pad(dst.reshape(2, 16, 25000), ((0, 0), (0, 0), (0, 600)),
                constant_values=N)
    return s.reshape(2, 16, ES_J, 128), d.reshape(2, 16, ES_J, 128)


def kernel(voxels, voxel_num_points, voxel_coords, edge_index, params):
    p = params
    src = edge_index[0].astype(jnp.int32)
    dst = edge_index[1].astype(jnp.int32)
    packs = {"cs": _pack_cs(src, dst), "es": _pack_es(src, dst)}
    agg_cs32 = _make_edge_agg(CS_J, 2 * N, 32, 20)
    agg_es32 = _make_edge_agg(ES_J, N, 32, 20)
    agg_es16 = _make_edge_agg(ES_J, N, 16, 40)

    def conv(xs, ws, variant="plain", extras=(), scale=None):
        cout = ws[0].shape[1]
        G = 16 if cout == 16 else 32
        nq = cout // G
        cins = tuple(x.shape[1] for x in xs)
        wstk = [jnp.stack([w[:, G * q:G * (q + 1)] for q in range(nq)])
                for w in ws]
        k1 = _make_k1(cins, cout, scaled=scale is not None)
        args = list(xs) + wstk + ([scale] if scale is not None else [])
        table = k1(*args)
        if cout == 64:
            spack, dpack = packs["cs"]
            agg = agg_cs32(table.reshape(2 * N, 32), spack, dpack)
        else:
            spack, dpack = packs["es"]
            agg_f = agg_es16 if cout == 16 else agg_es32
            agg = agg_f(table.reshape(N, G), spack, dpack)
        k3 = _make_k3(cout, variant)
        return k3(table, agg, *extras)

    def block(x, w):
        return conv([x], [w])

    def basic(x, wa, wb):
        h = conv([x], [wa])
        return conv([h], [wb], variant="resid", extras=(x,))

    def ur(x_lat, x_bot, wta, wtb, wm, winv):
        xt = basic(x_lat, wta, wtb)
        c = xt.shape[1]
        xm = conv([x_bot, xt], [wm[:c], wm[c:]], variant="ur",
                  extras=(x_bot, xt))
        return conv([xm], [winv])

    xv = voxels.reshape(N, 20)
    w_rep = jnp.concatenate([p['W_in']] * 5, axis=0)
    cnt = voxel_num_points.astype(jnp.float32).reshape(N, 1)
    x = conv([xv], [w_rep], scale=cnt)

    x1 = block(x, p['W1'])
    x2 = block(block(block(x1, p['W2a']), p['W2b']), p['W2c'])
    x3 = block(block(block(x2, p['W3a']), p['W3b']), p['W3c'])
    x4 = block(block(block(x3, p['W4a']), p['W4b']), p['W4c'])
    up4 = ur(x4, x4, p['Wt4a'], p['Wt4b'], p['Wm4'], p['Winv4'])
    up3 = ur(x3, up4, p['Wt3a'], p['Wt3b'], p['Wm3'], p['Winv3'])
    up2 = ur(x2, up3, p['Wt2a'], p['Wt2b'], p['Wm2'], p['Winv2'])
    up1 = ur(x1, up2, p['Wt1a'], p['Wt1b'], p['Wm1'], p['Wc5'])
    return up1
